# baseline (speedup 1.0000x reference)
import functools

import jax
import jax.numpy as jnp
from jax import lax
from jax.experimental import pallas as pl
from jax.experimental.pallas import tpu as pltpu
from jax.experimental.pallas import tpu_sc as plsc

N = 10000
E = 320000
D = 128
G = 64
NC = 2
NS = 16
NW = NC * NS
EPT = E // NW
KP = 128
EPP = 10240
CH = EPP // KP
NP = N + 16
RPT = N // NS

_mesh = plsc.VectorSubcoreMesh(core_axis_name="c", subcore_axis_name="s")


DW = D
NB = 2
NBD = 4


@functools.partial(
    pl.kernel,
    out_type=jax.ShapeDtypeStruct((NC, NS, RPT, DW), jnp.float32),
    mesh=_mesh,
    scratch_types=[
        pltpu.VMEM((KP, DW), jnp.float32),
        pltpu.VMEM((NBD, KP), jnp.int32),
        pltpu.VMEM_SHARED((NP, DW), jnp.float32),
        pltpu.SemaphoreType.DMA((NBD,)),
        pltpu.SemaphoreType.DMA((NBD,)),
    ],
)
def _deg_kernel(dst_hbm, ones_hbm, zero_hbm, out_hbm,
                ones_v, didx, acc_sh, dsem, ssem):
    cid = lax.axis_index("c")
    sid = lax.axis_index("s")
    wid = cid * NS + sid
    pltpu.sync_copy(ones_hbm, ones_v)
    pltpu.sync_copy(zero_hbm, acc_sh.at[pl.ds(sid * RPT, RPT)])
    plsc.subcore_barrier()

    for b in range(NBD):
        pltpu.async_copy(dst_hbm.at[wid, b], didx.at[b], dsem.at[b])

    def body(i, _):
        for b in range(NBD):
            pltpu.make_async_copy(dst_hbm.at[wid, 0], didx.at[b],
                                  dsem.at[b]).wait()
            pltpu.async_copy(ones_v, acc_sh.at[didx.at[b]], ssem.at[b],
                             add=True)
        for b in range(NBD):
            j = NBD * i + b
            pltpu.make_async_copy(ones_v, acc_sh.at[didx.at[b]],
                                  ssem.at[b]).wait()

            @pl.when(j + NBD < CH)
            def _():
                pltpu.async_copy(dst_hbm.at[wid, j + NBD], didx.at[b],
                                 dsem.at[b])
        return 0

    lax.fori_loop(0, CH // NBD, body, 0)
    plsc.subcore_barrier()
    pltpu.sync_copy(acc_sh.at[pl.ds(sid * RPT, RPT)], out_hbm.at[cid, sid])


@functools.partial(
    pl.kernel,
    out_type=jax.ShapeDtypeStruct((NC, NS, RPT, D), jnp.float32),
    mesh=_mesh,
    scratch_types=[
        pltpu.VMEM((NB, KP, D), jnp.float32),
        pltpu.VMEM((NB, KP), jnp.int32),
        pltpu.VMEM((NB, KP), jnp.int32),
        pltpu.VMEM_SHARED((NP, D), jnp.float32),
        pltpu.SemaphoreType.DMA((NB,)),
        pltpu.SemaphoreType.DMA((NB,)),
        pltpu.SemaphoreType.DMA((NB,)),
        pltpu.SemaphoreType.DMA((NB,)),
    ],
)
def _scatter_kernel(y_hbm, src_hbm, dst_hbm, zero_hbm, out_hbm,
                    rows_v, sidx, didx, acc_sh, gsem, ssem, srcsem, dsem):
    cid = lax.axis_index("c")
    sid = lax.axis_index("s")
    wid = cid * NS + sid
    pltpu.sync_copy(zero_hbm, acc_sh.at[pl.ds(sid * RPT, RPT)])
    plsc.subcore_barrier()

    for b in range(NB):
        pltpu.async_copy(src_hbm.at[wid, b], sidx.at[b], srcsem.at[b])
        pltpu.async_copy(dst_hbm.at[wid, b], didx.at[b], dsem.at[b])
    for b in range(NB):
        pltpu.make_async_copy(src_hbm.at[wid, 0], sidx.at[b],
                              srcsem.at[b]).wait()
        pltpu.async_copy(y_hbm.at[sidx.at[b]], rows_v.at[b], gsem.at[b])

    def body(i, _):
        for b in range(NB):
            j = NB * i + b
            pltpu.make_async_copy(y_hbm.at[sidx.at[b]], rows_v.at[b],
                                  gsem.at[b]).wait()
            pltpu.make_async_copy(dst_hbm.at[wid, 0], didx.at[b],
                                  dsem.at[b]).wait()
            pltpu.async_copy(rows_v.at[b], acc_sh.at[didx.at[b]],
                             ssem.at[b], add=True)

            @pl.when(j + NB < CH)
            def _():
                pltpu.async_copy(src_hbm.at[wid, j + NB], sidx.at[b],
                                 srcsem.at[b])
        for b in range(NB):
            j = NB * i + b
            pltpu.make_async_copy(rows_v.at[b], acc_sh.at[didx.at[b]],
                                  ssem.at[b]).wait()

            @pl.when(j + NB < CH)
            def _():
                pltpu.async_copy(dst_hbm.at[wid, j + NB], didx.at[b],
                                 dsem.at[b])
                pltpu.make_async_copy(src_hbm.at[wid, 0], sidx.at[b],
                                      srcsem.at[b]).wait()
                pltpu.async_copy(y_hbm.at[sidx.at[b]], rows_v.at[b],
                                 gsem.at[b])
        return 0

    lax.fori_loop(0, CH // NB, body, 0)
    plsc.subcore_barrier()
    pltpu.sync_copy(acc_sh.at[pl.ds(sid * RPT, RPT)], out_hbm.at[cid, sid])


_R = 1000
_NBLK = N // _R


def _tdinv_body(hist_ref, o_ref):
    deg = hist_ref[0, :, 0:1] + hist_ref[1, :, 0:1] + 1.0
    o_ref[...] = lax.rsqrt(deg)


def _t1_body(dinv_ref, x_ref, w_ref, o_ref):
    z = jnp.dot(x_ref[...], w_ref[...], preferred_element_type=jnp.float32)
    o_ref[...] = z * dinv_ref[...]


def _t2_body(p_ref, y_ref, dinv_ref, b_ref, w_ref, o_ref):
    dinv = dinv_ref[...]
    s = p_ref[0] + p_ref[1] + y_ref[...]
    h = jnp.maximum(s * dinv + b_ref[...], 0.0)
    o_ref[...] = jnp.dot(h, w_ref[...],
                         preferred_element_type=jnp.float32) * dinv


def _t3_body(p_ref, y_ref, dinv_ref, b_ref, batch_ref,
             osum_ref, omax_ref, ocnt_ref):
    i = pl.program_id(0)
    s = p_ref[0] + p_ref[1] + y_ref[...]
    h = jnp.maximum(s * dinv_ref[...] + b_ref[...], 0.0)
    bt_col = batch_ref[0]

    @pl.when(i == 0)
    def _():
        osum_ref[...] = jnp.zeros_like(osum_ref)
        omax_ref[...] = jnp.full_like(omax_ref, -jnp.inf)
        ocnt_ref[...] = jnp.zeros_like(ocnt_ref)

    seg = lax.broadcasted_iota(jnp.int32, (_R, G), 1)
    onehot = (seg == bt_col).astype(jnp.float32)
    contract0 = (((0,), (0,)), ((), ()))
    osum_ref[...] += lax.dot_general(
        onehot, h, contract0, preferred_element_type=jnp.float32)
    ocnt_ref[...] += lax.dot_general(
        onehot, jnp.ones((_R, 1), jnp.float32), contract0,
        preferred_element_type=jnp.float32)

    def mbody(g, _):
        m = jnp.max(jnp.where(bt_col == g, h, -jnp.inf),
                    axis=0, keepdims=True)
        omax_ref[pl.ds(g, 1), :] = jnp.maximum(omax_ref[pl.ds(g, 1), :], m)
        return 0

    lax.fori_loop(batch_ref[0, 0, 0], batch_ref[0, _R - 1, 0] + 1, mbody, 0)


def _t4_body(sum_ref, max_ref, cnt_ref, m1a_ref, m1b_ref, mb1_ref,
             m2_ref, mb2_ref, m3_ref, mb3_ref, o_ref):
    mean = sum_ref[...] / jnp.maximum(cnt_ref[...], 1.0)
    g1 = jnp.maximum(
        jnp.dot(mean, m1a_ref[...], preferred_element_type=jnp.float32)
        + jnp.dot(max_ref[...], m1b_ref[...], preferred_element_type=jnp.float32)
        + mb1_ref[...], 0.0)
    g2 = jnp.maximum(
        jnp.dot(g1, m2_ref[...], preferred_element_type=jnp.float32)
        + mb2_ref[...], 0.0)
    g3 = jnp.dot(g2, m3_ref[...], preferred_element_type=jnp.float32) + mb3_ref[...]
    o_ref[...] = jax.nn.sigmoid(g3)


def _hist_spec():
    return pl.BlockSpec((NC, _R, DW), lambda i: (0, i, 0))


def _dinv_spec():
    return pl.BlockSpec((_R, 1), lambda i: (i, 0))


def _tdinv(hist):
    return pl.pallas_call(
        _tdinv_body,
        grid=(_NBLK,),
        in_specs=[_hist_spec()],
        out_specs=_dinv_spec(),
        out_shape=jax.ShapeDtypeStruct((N, 1), jnp.float32),
    )(hist)


def _rows_spec():
    return pl.BlockSpec((_R, D), lambda i: (i, 0))


def _full_spec(shape):
    nd = len(shape)
    return pl.BlockSpec(shape, lambda *a: (0,) * nd)


def _t1(dinv, x, w1):
    return pl.pallas_call(
        _t1_body,
        grid=(_NBLK,),
        in_specs=[_dinv_spec(), _rows_spec(), _full_spec((D, D))],
        out_specs=_rows_spec(),
        out_shape=jax.ShapeDtypeStruct((N, D), jnp.float32),
    )(dinv, x, w1)


def _t2(p, y, dinv, b, w):
    return pl.pallas_call(
        _t2_body,
        grid=(_NBLK,),
        in_specs=[
            pl.BlockSpec((NC, _R, D), lambda i: (0, i, 0)),
            _rows_spec(), _dinv_spec(), _full_spec((1, D)), _full_spec((D, D)),
        ],
        out_specs=_rows_spec(),
        out_shape=jax.ShapeDtypeStruct((N, D), jnp.float32),
    )(p, y, dinv, b, w)


def _t3(p, y, dinv, b, batch3):
    return pl.pallas_call(
        _t3_body,
        grid=(_NBLK,),
        in_specs=[
            pl.BlockSpec((NC, _R, D), lambda i: (0, i, 0)),
            _rows_spec(), _dinv_spec(), _full_spec((1, D)),
            pl.BlockSpec((1, _R, 1), lambda i: (i, 0, 0)),
        ],
        out_specs=[_full_spec((G, D)), _full_spec((G, D)),
                   _full_spec((G, 1))],
        out_shape=[jax.ShapeDtypeStruct((G, D), jnp.float32),
                   jax.ShapeDtypeStruct((G, D), jnp.float32),
                   jax.ShapeDtypeStruct((G, 1), jnp.float32)],
    )(p, y, dinv, b, batch3)


def _t4(seg_sum, seg_max, seg_cnt, m1a, m1b, mb1, m2, mb2, m3, mb3):
    return pl.pallas_call(
        _t4_body,
        in_specs=[_full_spec((G, D)), _full_spec((G, D)), _full_spec((G, 1)),
                  _full_spec((D, D)), _full_spec((D, D)), _full_spec((1, D)),
                  _full_spec((D, D // 2)), _full_spec((1, D // 2)),
                  _full_spec((D // 2, 1)), _full_spec((1, 1))],
        out_specs=_full_spec((G, 1)),
        out_shape=jax.ShapeDtypeStruct((G, 1), jnp.float32),
    )(seg_sum, seg_max, seg_cnt, m1a, m1b, mb1, m2, mb2, m3, mb3)


def kernel(x, edge_index, edge_attr, batch, W1, b1, W2, b2, W3, b3,
           M1, mb1, M2, mb2, M3, mb3):
    del edge_attr
    src = edge_index[0].astype(jnp.int32).reshape(NW, EPT)
    dst = edge_index[1].astype(jnp.int32).reshape(NW, EPT)
    pad_s = jnp.zeros((NW, EPP - EPT), jnp.int32)
    pad_d = jnp.full((NW, EPP - EPT), N, jnp.int32)
    src = jnp.concatenate([src, pad_s], axis=1).reshape(NW, CH, KP)
    dst = jnp.concatenate([dst, pad_d], axis=1).reshape(NW, CH, KP)
    batch3 = batch.astype(jnp.int32).reshape(_NBLK, _R, 1)
    zeros = jnp.zeros((RPT, D), jnp.float32)

    hist = _deg_kernel(dst, jnp.ones((KP, DW), jnp.float32),
                       zeros).reshape(NC, N, DW)
    dinv = _tdinv(hist)

    def _scatter(y):
        return _scatter_kernel(y, src, dst, zeros).reshape(NC, N, D)

    y1 = _t1(dinv, x, W1)
    p1 = _scatter(y1)
    y2 = _t2(p1, y1, dinv, b1.reshape(1, D), W2)
    p2 = _scatter(y2)
    y3 = _t2(p2, y2, dinv, b2.reshape(1, D), W3)
    p3 = _scatter(y3)
    seg_sum, seg_max, seg_cnt = _t3(p3, y3, dinv, b3.reshape(1, D), batch3)

    return _t4(seg_sum, seg_max, seg_cnt,
               M1[:D], M1[D:], mb1.reshape(1, D),
               M2, mb2.reshape(1, D // 2), M3, mb3.reshape(1, 1))


# speedup vs baseline: 2.2588x; 2.2588x over previous
import functools

import jax
import jax.numpy as jnp
from jax import lax
from jax.experimental import pallas as pl
from jax.experimental.pallas import tpu as pltpu
from jax.experimental.pallas import tpu_sc as plsc

N = 10000
E = 320000
D = 128
G = 64
NC = 2
NS = 16
NW = NC * NS
EPT = E // NW
KP = 80
CH = EPT // KP
RPT = N // NS

_mesh = plsc.VectorSubcoreMesh(core_axis_name="c", subcore_axis_name="s")


DW = D


@functools.partial(
    pl.kernel,
    out_type=jax.ShapeDtypeStruct((NC, NS, RPT, DW), jnp.float32),
    mesh=_mesh,
    scratch_types=[
        pltpu.VMEM((KP, DW), jnp.float32),
        pltpu.VMEM((EPT,), jnp.int32),
        pltpu.VMEM_SHARED((N, DW), jnp.float32),
        pltpu.SemaphoreType.DMA,
        pltpu.SemaphoreType.DMA,
    ],
)
def _deg_kernel(dst_hbm, ones_hbm, zero_hbm, out_hbm,
                ones_v, didx, acc_sh, sem0, sem1):
    cid = lax.axis_index("c")
    sid = lax.axis_index("s")
    wid = cid * NS + sid
    pltpu.sync_copy(ones_hbm, ones_v)
    pltpu.sync_copy(dst_hbm.at[pl.ds(wid * EPT, EPT)], didx)
    pltpu.sync_copy(zero_hbm, acc_sh.at[pl.ds(sid * RPT, RPT)])
    plsc.subcore_barrier()

    def body(i, _):
        j = 2 * i
        s0 = pltpu.async_copy(
            ones_v, acc_sh.at[didx.at[pl.ds(j * KP, KP)]], sem0, add=True)
        s1 = pltpu.async_copy(
            ones_v, acc_sh.at[didx.at[pl.ds((j + 1) * KP, KP)]], sem1,
            add=True)
        s0.wait()
        s1.wait()
        return 0

    lax.fori_loop(0, CH // 2, body, 0)
    pltpu.async_copy(ones_v, acc_sh.at[didx.at[pl.ds((CH - 1) * KP, KP)]],
                     sem0, add=True).wait()
    plsc.subcore_barrier()
    pltpu.sync_copy(acc_sh.at[pl.ds(sid * RPT, RPT)], out_hbm.at[cid, sid])


@functools.partial(
    pl.kernel,
    out_type=jax.ShapeDtypeStruct((NC, NS, RPT, D), jnp.float32),
    mesh=_mesh,
    scratch_types=[
        pltpu.VMEM((KP, D), jnp.float32),
        pltpu.VMEM((KP, D), jnp.float32),
        pltpu.VMEM((EPT,), jnp.int32),
        pltpu.VMEM((EPT,), jnp.int32),
        pltpu.VMEM_SHARED((N, D), jnp.float32),
        pltpu.SemaphoreType.DMA,
        pltpu.SemaphoreType.DMA,
        pltpu.SemaphoreType.DMA,
        pltpu.SemaphoreType.DMA,
    ],
)
def _scatter_kernel(y_hbm, src_hbm, dst_hbm, zero_hbm, out_hbm,
                    rows_a, rows_b, sidx, didx, acc_sh,
                    gsem0, gsem1, ssem0, ssem1):
    cid = lax.axis_index("c")
    sid = lax.axis_index("s")
    wid = cid * NS + sid
    pltpu.sync_copy(src_hbm.at[pl.ds(wid * EPT, EPT)], sidx)
    pltpu.sync_copy(dst_hbm.at[pl.ds(wid * EPT, EPT)], didx)
    pltpu.sync_copy(zero_hbm, acc_sh.at[pl.ds(sid * RPT, RPT)])
    plsc.subcore_barrier()

    def body(i, _):
        j = 2 * i
        g0 = pltpu.async_copy(
            y_hbm.at[sidx.at[pl.ds(j * KP, KP)]], rows_a, gsem0)
        g1 = pltpu.async_copy(
            y_hbm.at[sidx.at[pl.ds((j + 1) * KP, KP)]], rows_b, gsem1)
        g0.wait()
        s0 = pltpu.async_copy(
            rows_a, acc_sh.at[didx.at[pl.ds(j * KP, KP)]], ssem0, add=True)
        g1.wait()
        s1 = pltpu.async_copy(
            rows_b, acc_sh.at[didx.at[pl.ds((j + 1) * KP, KP)]], ssem1,
            add=True)
        s0.wait()
        s1.wait()
        return 0

    lax.fori_loop(0, CH // 2, body, 0)
    j = CH - 1
    pltpu.async_copy(y_hbm.at[sidx.at[pl.ds(j * KP, KP)]], rows_a,
                     gsem0).wait()
    pltpu.async_copy(rows_a, acc_sh.at[didx.at[pl.ds(j * KP, KP)]],
                     ssem0, add=True).wait()
    plsc.subcore_barrier()
    pltpu.sync_copy(acc_sh.at[pl.ds(sid * RPT, RPT)], out_hbm.at[cid, sid])


_R = 1000
_NBLK = N // _R


def _tdinv_body(hist_ref, o_ref):
    deg = hist_ref[0, :, 0:1] + hist_ref[1, :, 0:1] + 1.0
    o_ref[...] = lax.rsqrt(deg)


def _t1_body(dinv_ref, x_ref, w_ref, o_ref):
    z = jnp.dot(x_ref[...], w_ref[...], preferred_element_type=jnp.float32)
    o_ref[...] = z * dinv_ref[...]


def _t2_body(p_ref, y_ref, dinv_ref, b_ref, w_ref, o_ref):
    dinv = dinv_ref[...]
    s = p_ref[0] + p_ref[1] + y_ref[...]
    h = jnp.maximum(s * dinv + b_ref[...], 0.0)
    o_ref[...] = jnp.dot(h, w_ref[...],
                         preferred_element_type=jnp.float32) * dinv


def _t3_body(p_ref, y_ref, dinv_ref, b_ref, batch_ref,
             osum_ref, omax_ref, ocnt_ref):
    i = pl.program_id(0)
    s = p_ref[0] + p_ref[1] + y_ref[...]
    h = jnp.maximum(s * dinv_ref[...] + b_ref[...], 0.0)
    bt_col = batch_ref[0]

    @pl.when(i == 0)
    def _():
        osum_ref[...] = jnp.zeros_like(osum_ref)
        omax_ref[...] = jnp.full_like(omax_ref, -jnp.inf)
        ocnt_ref[...] = jnp.zeros_like(ocnt_ref)

    seg = lax.broadcasted_iota(jnp.int32, (_R, G), 1)
    onehot = (seg == bt_col).astype(jnp.float32)
    contract0 = (((0,), (0,)), ((), ()))
    osum_ref[...] += lax.dot_general(
        onehot, h, contract0, preferred_element_type=jnp.float32)
    ocnt_ref[...] += lax.dot_general(
        onehot, jnp.ones((_R, 1), jnp.float32), contract0,
        preferred_element_type=jnp.float32)

    def mbody(g, _):
        m = jnp.max(jnp.where(bt_col == g, h, -jnp.inf),
                    axis=0, keepdims=True)
        omax_ref[pl.ds(g, 1), :] = jnp.maximum(omax_ref[pl.ds(g, 1), :], m)
        return 0

    lax.fori_loop(batch_ref[0, 0, 0], batch_ref[0, _R - 1, 0] + 1, mbody, 0)


def _t4_body(sum_ref, max_ref, cnt_ref, m1a_ref, m1b_ref, mb1_ref,
             m2_ref, mb2_ref, m3_ref, mb3_ref, o_ref):
    mean = sum_ref[...] / jnp.maximum(cnt_ref[...], 1.0)
    g1 = jnp.maximum(
        jnp.dot(mean, m1a_ref[...], preferred_element_type=jnp.float32)
        + jnp.dot(max_ref[...], m1b_ref[...], preferred_element_type=jnp.float32)
        + mb1_ref[...], 0.0)
    g2 = jnp.maximum(
        jnp.dot(g1, m2_ref[...], preferred_element_type=jnp.float32)
        + mb2_ref[...], 0.0)
    g3 = jnp.dot(g2, m3_ref[...], preferred_element_type=jnp.float32) + mb3_ref[...]
    o_ref[...] = jax.nn.sigmoid(g3)


def _hist_spec():
    return pl.BlockSpec((NC, _R, DW), lambda i: (0, i, 0))


def _dinv_spec():
    return pl.BlockSpec((_R, 1), lambda i: (i, 0))


def _tdinv(hist):
    return pl.pallas_call(
        _tdinv_body,
        grid=(_NBLK,),
        in_specs=[_hist_spec()],
        out_specs=_dinv_spec(),
        out_shape=jax.ShapeDtypeStruct((N, 1), jnp.float32),
    )(hist)


def _rows_spec():
    return pl.BlockSpec((_R, D), lambda i: (i, 0))


def _full_spec(shape):
    nd = len(shape)
    return pl.BlockSpec(shape, lambda *a: (0,) * nd)


def _t1(dinv, x, w1):
    return pl.pallas_call(
        _t1_body,
        grid=(_NBLK,),
        in_specs=[_dinv_spec(), _rows_spec(), _full_spec((D, D))],
        out_specs=_rows_spec(),
        out_shape=jax.ShapeDtypeStruct((N, D), jnp.float32),
    )(dinv, x, w1)


def _t2(p, y, dinv, b, w):
    return pl.pallas_call(
        _t2_body,
        grid=(_NBLK,),
        in_specs=[
            pl.BlockSpec((NC, _R, D), lambda i: (0, i, 0)),
            _rows_spec(), _dinv_spec(), _full_spec((1, D)), _full_spec((D, D)),
        ],
        out_specs=_rows_spec(),
        out_shape=jax.ShapeDtypeStruct((N, D), jnp.float32),
    )(p, y, dinv, b, w)


def _t3(p, y, dinv, b, batch3):
    return pl.pallas_call(
        _t3_body,
        grid=(_NBLK,),
        in_specs=[
            pl.BlockSpec((NC, _R, D), lambda i: (0, i, 0)),
            _rows_spec(), _dinv_spec(), _full_spec((1, D)),
            pl.BlockSpec((1, _R, 1), lambda i: (i, 0, 0)),
        ],
        out_specs=[_full_spec((G, D)), _full_spec((G, D)),
                   _full_spec((G, 1))],
        out_shape=[jax.ShapeDtypeStruct((G, D), jnp.float32),
                   jax.ShapeDtypeStruct((G, D), jnp.float32),
                   jax.ShapeDtypeStruct((G, 1), jnp.float32)],
    )(p, y, dinv, b, batch3)


def _t4(seg_sum, seg_max, seg_cnt, m1a, m1b, mb1, m2, mb2, m3, mb3):
    return pl.pallas_call(
        _t4_body,
        in_specs=[_full_spec((G, D)), _full_spec((G, D)), _full_spec((G, 1)),
                  _full_spec((D, D)), _full_spec((D, D)), _full_spec((1, D)),
                  _full_spec((D, D // 2)), _full_spec((1, D // 2)),
                  _full_spec((D // 2, 1)), _full_spec((1, 1))],
        out_specs=_full_spec((G, 1)),
        out_shape=jax.ShapeDtypeStruct((G, 1), jnp.float32),
    )(seg_sum, seg_max, seg_cnt, m1a, m1b, mb1, m2, mb2, m3, mb3)


def kernel(x, edge_index, edge_attr, batch, W1, b1, W2, b2, W3, b3,
           M1, mb1, M2, mb2, M3, mb3):
    del edge_attr
    src = edge_index[0].astype(jnp.int32)
    dst = edge_index[1].astype(jnp.int32)
    batch3 = batch.astype(jnp.int32).reshape(_NBLK, _R, 1)
    zeros = jnp.zeros((RPT, D), jnp.float32)

    hist = _deg_kernel(dst, jnp.ones((KP, DW), jnp.float32),
                       zeros).reshape(NC, N, DW)
    dinv = _tdinv(hist)

    def _scatter(y):
        return _scatter_kernel(y, src, dst, zeros).reshape(NC, N, D)

    y1 = _t1(dinv, x, W1)
    p1 = _scatter(y1)
    y2 = _t2(p1, y1, dinv, b1.reshape(1, D), W2)
    p2 = _scatter(y2)
    y3 = _t2(p2, y2, dinv, b2.reshape(1, D), W3)
    p3 = _scatter(y3)
    seg_sum, seg_max, seg_cnt = _t3(p3, y3, dinv, b3.reshape(1, D), batch3)

    return _t4(seg_sum, seg_max, seg_cnt,
               M1[:D], M1[D:], mb1.reshape(1, D),
               M2, mb2.reshape(1, D // 2), M3, mb3.reshape(1, 1))
